# baseline (speedup 1.0000x reference)
import functools

import jax
import jax.numpy as jnp
from jax import lax
from jax.experimental import pallas as pl
from jax.experimental.pallas import tpu as pltpu
from jax.experimental.pallas import tpu_sc as plsc

_NUM_CORES = 2
_NUM_SUBCORES = 16
_NW = _NUM_CORES * _NUM_SUBCORES
_NBUF = 5
_NTR = 3


def kernel(indices, table):
    B, L = indices.shape
    V, D = table.shape
    per_w = B // _NW
    n_groups = L // _NBUF

    idx_t = indices.T.astype(jnp.int32)
    tab_p = jnp.pad(table, ((0, 0), (0, 128 - D)))

    mesh = plsc.VectorSubcoreMesh(core_axis_name="c", subcore_axis_name="s")

    @functools.partial(
        pl.kernel,
        out_type=jax.ShapeDtypeStruct((L, D, B), jnp.float32),
        mesh=mesh,
        compiler_params=pltpu.CompilerParams(
            use_tc_tiling_on_sc=True, needs_layout_passes=False),
        scratch_types=[
            pltpu.VMEM((L, per_w), jnp.int32),
            pltpu.VMEM((_NBUF, per_w, 128), jnp.float32),
            pltpu.VMEM((_NTR, D, per_w), jnp.float32),
            [pltpu.SemaphoreType.DMA] * _NBUF,
            [pltpu.SemaphoreType.DMA] * _NTR,
        ],
    )
    def gather_kernel(idx_hbm, tab_hbm, out_hbm, idx_v, rows_v, trans_v,
                      gsems, wsems):
        wid = lax.axis_index("s") * _NUM_CORES + lax.axis_index("c")
        b0 = wid * per_w

        pltpu.sync_copy(idx_hbm.at[:, pl.ds(b0, per_w)], idx_v)

        lane = lax.iota(jnp.int32, 16)

        def start_gather(l, s):
            pltpu.async_copy(tab_hbm.at[idx_v.at[l]], rows_v.at[s], gsems[s])

        def wait_gather(l, s):
            pltpu.make_async_copy(
                tab_hbm.at[idx_v.at[l]], rows_v.at[s], gsems[s]).wait()

        def out_dst(l):
            return out_hbm.at[l, :, pl.ds(b0, per_w)]

        def start_write(l, t):
            pltpu.async_copy(trans_v.at[t], out_dst(l), wsems[t])

        def wait_write(l, t):
            pltpu.make_async_copy(trans_v.at[t], out_dst(l), wsems[t]).wait()

        def transpose(s, t):
            def body_d(d, carry):
                col = jnp.zeros((16,), jnp.int32) + d
                for j in range(per_w // 16):
                    v = plsc.load_gather(rows_v.at[s], [lane + 16 * j, col])
                    trans_v[t, d, pl.ds(16 * j, 16)] = v
                return carry
            lax.fori_loop(0, D, body_d, 0)

        def job(l, s, t, first):
            wait_gather(l, s)
            if not first:
                wait_write(l - _NTR, t)
            transpose(s, t)
            start_write(l, t)
            if isinstance(l, int):
                if l + _NBUF < L:
                    start_gather(l + _NBUF, s)
            else:
                start_gather(l + _NBUF, s)

        for s in range(_NBUF):
            start_gather(s, s)

        for b in range(_NBUF):
            job(b, b, b % _NTR, first=(b < _NTR))

        def outer(g, carry):
            for b in range(_NBUF):
                job(g * _NBUF + b, b, b % _NTR, first=False)
            return carry

        lax.fori_loop(1, n_groups - 1, outer, 0)

        g = n_groups - 1
        for b in range(_NBUF):
            l = g * _NBUF + b
            wait_gather(l, b)
            wait_write(l - _NTR, b % _NTR)
            transpose(b, b % _NTR)
            start_write(l, b % _NTR)
        for l in range(L - _NTR, L):
            wait_write(l, (l % _NBUF) % _NTR)

    out = gather_kernel(idx_t, tab_p)
    return jnp.transpose(out, (2, 0, 1))
